# baseline (speedup 1.0000x reference)
import functools

import jax
import jax.numpy as jnp
from jax import lax
from jax.experimental import pallas as pl
from jax.experimental.pallas import tpu as pltpu
from jax.experimental.pallas import tpu_sc as plsc

B, L, F = 4096, 200, 16
EMBED_DIM = 32
OUT_F = F - 1 + EMBED_DIM

NC, NS, LANES = 2, 16, 16
NW = NC * NS
LT = L // 8
NLB = 4
NBLK = L // NLB
CR = NLB * 128


def _sc_body(x_hbm, table_hbm, out_hbm,
             x_v, keys_v, emb_v, out_v, sem_in, sem_emb, sem_out):
    wid = lax.axis_index("s") * NC + lax.axis_index("c")
    lane = lax.iota(jnp.int32, LANES)

    def in_copy(bi, d):
        return pltpu.make_async_copy(
            x_hbm.at[pl.ds(bi * NLB, NLB), :, wid, :, :], x_v.at[d],
            sem_in.at[d])

    def out_copy(bi, d):
        l0 = bi * NLB
        return pltpu.make_async_copy(
            out_v.at[d],
            out_hbm.at[:, l0 // 8, wid, pl.ds(l0 % 8, NLB), :],
            sem_out.at[d])

    in_copy(0, 0).start()

    @pl.loop(0, NBLK // 2)
    def blk_pair(i):
        for d in range(2):
            bi = i * 2 + d

            in_copy(bi, d).wait()

            @pl.when(bi + 1 < NBLK)
            def _():
                in_copy(bi + 1, 1 - d).start()

            for li in range(NLB):
                for k in range(8):
                    kf = x_v[d, li, 1, 7, pl.ds(k * LANES, LANES)]
                    keys_v[d, li, pl.ds(k * LANES, LANES)] = (
                        kf.astype(jnp.int32))

            copies = [
                pltpu.make_async_copy(
                    table_hbm.at[keys_v.at[d, li]],
                    emb_v.at[d, pl.ds(li * 128, 128), :],
                    sem_emb.at[d],
                )
                for li in range(NLB)
            ]
            for cp in copies:
                cp.start()
            for cp in copies:
                cp.wait()

            @pl.when(bi >= 2)
            def _():
                out_copy(bi - 2, d).wait()

            @plsc.parallel_loop(0, F - 1, unroll=5)
            def timing_plane(c):
                tf = c // 8
                s = c % 8
                for li in range(NLB):
                    for k in range(8):
                        out_v[d, c, li, pl.ds(k * LANES, LANES)] = (
                            x_v[d, li, tf, s, pl.ds(k * LANES, LANES)])

            @plsc.parallel_loop(F - 1, OUT_F, unroll=8)
            def emb_plane(c):
                e = c - (F - 1)
                for li in range(NLB):
                    for k in range(8):
                        rows = li * 128 + k * LANES + lane
                        vals = plsc.load_gather(
                            emb_v.at[d],
                            [rows, jnp.full((LANES,), e, jnp.int32)])
                        out_v[d, c, li, pl.ds(k * LANES, LANES)] = vals

            out_copy(bi, d).start()

    out_copy(NBLK - 2, 0).wait()
    out_copy(NBLK - 1, 1).wait()


@jax.jit
def _sc_call(x5, table):
    mesh = plsc.VectorSubcoreMesh(core_axis_name="c", subcore_axis_name="s")
    return pl.kernel(
        _sc_body,
        out_type=jax.ShapeDtypeStruct((OUT_F, LT, NW, 8, 128), jnp.float32),
        mesh=mesh,
        scratch_types=[
            pltpu.VMEM((2, NLB, 2, 8, 128), jnp.float32),
            pltpu.VMEM((2, NLB, 128), jnp.int32),
            pltpu.VMEM((2, CR, EMBED_DIM), jnp.float32),
            pltpu.VMEM((2, OUT_F, NLB, 128), jnp.float32),
            pltpu.SemaphoreType.DMA((2,)),
            pltpu.SemaphoreType.DMA((2,)),
            pltpu.SemaphoreType.DMA((2,)),
        ],
        compiler_params=pltpu.CompilerParams(
            needs_layout_passes=False, use_tc_tiling_on_sc=False),
    )(x5, table)


def kernel(x, table):
    x5 = x.transpose(1, 2, 0).reshape(L, 2, 8, 32, 128).transpose(0, 1, 3, 2, 4)
    out5 = _sc_call(x5, table)
    out = out5.transpose(0, 1, 3, 2, 4).reshape(OUT_F, L, B).transpose(2, 1, 0)
    return out


# speedup vs baseline: 1.1059x; 1.1059x over previous
import functools

import jax
import jax.numpy as jnp
from jax import lax
from jax.experimental import pallas as pl
from jax.experimental.pallas import tpu as pltpu
from jax.experimental.pallas import tpu_sc as plsc

B, L, F = 4096, 200, 16
EMBED_DIM = 32
OUT_F = F - 1 + EMBED_DIM

NC, NS, LANES = 2, 16, 16
NW = NC * NS
LT = L // 8
NLB = 4
NBLK = L // NLB
CR = NLB * 128


def _sc_body(x_hbm, table_hbm, out_hbm,
             x_v, keys_v, emb_v, out_v, sem_in, sem_emb, sem_out):
    wid = lax.axis_index("s") * NC + lax.axis_index("c")
    lane = lax.iota(jnp.int32, LANES)

    def in_copy(bi, d):
        return pltpu.make_async_copy(
            x_hbm.at[pl.ds(bi * NLB, NLB), :, wid, :, :], x_v.at[d],
            sem_in.at[d])

    def out_copy(bi, d):
        l0 = bi * NLB
        return pltpu.make_async_copy(
            out_v.at[d],
            out_hbm.at[:, l0 // 8, wid, pl.ds(l0 % 8, NLB), :],
            sem_out.at[d])

    def extract_and_gather(bi, d):
        for li in range(NLB):
            for k in range(8):
                kf = x_v[d, li, 1, 7, pl.ds(k * LANES, LANES)]
                keys_v[d, li, pl.ds(k * LANES, LANES)] = kf.astype(jnp.int32)
        for li in range(NLB):
            pltpu.make_async_copy(
                table_hbm.at[keys_v.at[d, li]],
                emb_v.at[d, pl.ds(li * 128, 128), :],
                sem_emb.at[d],
            ).start()

    def assemble(bi, d):
        pltpu.make_async_copy(
            table_hbm.at[keys_v.at[d, 0]],
            emb_v.at[d, pl.ds(0, 128), :], sem_emb.at[d],
        ).wait()
        for li in range(1, NLB):
            pltpu.make_async_copy(
                table_hbm.at[keys_v.at[d, li]],
                emb_v.at[d, pl.ds(li * 128, 128), :], sem_emb.at[d],
            ).wait()

        @pl.when(bi >= 2)
        def _():
            out_copy(bi - 2, d).wait()

        @plsc.parallel_loop(0, F - 1, unroll=5)
        def timing_plane(c):
            tf = c // 8
            s = c % 8
            for li in range(NLB):
                for k in range(8):
                    out_v[d, c, li, pl.ds(k * LANES, LANES)] = (
                        x_v[d, li, tf, s, pl.ds(k * LANES, LANES)])

        @plsc.parallel_loop(F - 1, OUT_F, unroll=4)
        def emb_plane(c):
            e = c - (F - 1)
            for li in range(NLB):
                for k in range(8):
                    rows = li * 128 + k * LANES + lane
                    vals = plsc.load_gather(
                        emb_v.at[d],
                        [rows, jnp.full((LANES,), e, jnp.int32)])
                    out_v[d, c, li, pl.ds(k * LANES, LANES)] = vals

        out_copy(bi, d).start()

    in_copy(0, 0).start()

    @pl.loop(0, NBLK // 2)
    def blk_pair(i):
        for d in range(2):
            bi = i * 2 + d
            in_copy(bi, d).wait()
            extract_and_gather(bi, d)

            @pl.when(bi >= 1)
            def _():
                assemble(bi - 1, 1 - d)

            @pl.when(bi + 1 < NBLK)
            def _():
                in_copy(bi + 1, 1 - d).start()

    assemble(NBLK - 1, (NBLK - 1) % 2)
    out_copy(NBLK - 2, 0).wait()
    out_copy(NBLK - 1, 1).wait()


@jax.jit
def _sc_call(x5, table):
    mesh = plsc.VectorSubcoreMesh(core_axis_name="c", subcore_axis_name="s")
    return pl.kernel(
        _sc_body,
        out_type=jax.ShapeDtypeStruct((OUT_F, LT, NW, 8, 128), jnp.float32),
        mesh=mesh,
        scratch_types=[
            pltpu.VMEM((2, NLB, 2, 8, 128), jnp.float32),
            pltpu.VMEM((2, NLB, 128), jnp.int32),
            pltpu.VMEM((2, CR, EMBED_DIM), jnp.float32),
            pltpu.VMEM((2, OUT_F, NLB, 128), jnp.float32),
            pltpu.SemaphoreType.DMA((2,)),
            pltpu.SemaphoreType.DMA((2,)),
            pltpu.SemaphoreType.DMA((2,)),
        ],
        compiler_params=pltpu.CompilerParams(
            needs_layout_passes=False, use_tc_tiling_on_sc=False),
    )(x5, table)


def kernel(x, table):
    x5 = x.transpose(1, 2, 0).reshape(L, 2, 8, 32, 128).transpose(0, 1, 3, 2, 4)
    out5 = _sc_call(x5, table)
    out = out5.transpose(0, 1, 3, 2, 4).reshape(OUT_F, L, B).transpose(2, 1, 0)
    return out
